# Initial kernel scaffold; baseline (speedup 1.0000x reference)
import jax
import jax.numpy as jnp
from jax.experimental import pallas as pl


def kernel(feat, edge_index):
    raise NotImplementedError("write your pallas kernel here")


# speedup vs baseline: 4.1555x; 4.1555x over previous
import functools

import jax
import jax.numpy as jnp
from jax import lax
from jax.experimental import pallas as pl
from jax.experimental.pallas import tpu as pltpu
from jax.experimental.pallas import tpu_sc as plsc

NC = 2
NS = 16
NW = NC * NS
K = 128

T0, T1, T2 = 0.6, -0.4, 0.2


def _degree_sc(src3, zeros_vec, ones_vec, r_pad):
    nb = src3.shape[1]
    rows_per = r_pad // NS
    mesh = plsc.VectorSubcoreMesh(core_axis_name="c", subcore_axis_name="s")

    @functools.partial(
        pl.kernel,
        out_type=jax.ShapeDtypeStruct((NC * r_pad,), jnp.float32),
        mesh=mesh,
        scratch_types=[
            pltpu.VMEM((nb, K), jnp.int32),
            pltpu.VMEM((K,), jnp.float32),
            pltpu.VMEM_SHARED((r_pad,), jnp.float32),
        ],
    )
    def deg_kernel(src_hbm, z_hbm, ones_hbm, out_hbm, src_v, ones_v, acc):
        c = lax.axis_index("c")
        s = lax.axis_index("s")
        wid = c * NS + s
        base = s * rows_per
        pltpu.sync_copy(src_hbm.at[wid], src_v)
        pltpu.sync_copy(ones_hbm, ones_v)
        pltpu.sync_copy(z_hbm.at[pl.ds(base, rows_per)], acc.at[pl.ds(base, rows_per)])
        plsc.subcore_barrier()

        def body(j, carry):
            pltpu.sync_copy(ones_v, acc.at[src_v.at[j]], add=True)
            return carry

        lax.fori_loop(0, nb, body, 0)
        plsc.subcore_barrier()
        pltpu.sync_copy(acc.at[pl.ds(base, rows_per)],
                        out_hbm.at[pl.ds(c * r_pad + base, rows_per)])

    return deg_kernel(src3, zeros_vec, ones_vec)


def _segsum_sc(h, src3, dst3, zeros_rows, r_pad):
    nb = src3.shape[1]
    d = h.shape[1]
    rows_per = r_pad // NS
    mesh = plsc.VectorSubcoreMesh(core_axis_name="c", subcore_axis_name="s")

    @functools.partial(
        pl.kernel,
        out_type=jax.ShapeDtypeStruct((NC, r_pad, d), jnp.float32),
        mesh=mesh,
        scratch_types=[
            pltpu.VMEM((nb, K), jnp.int32),
            pltpu.VMEM((nb, K), jnp.int32),
            pltpu.VMEM((K, d), jnp.float32),
            pltpu.VMEM_SHARED((r_pad, d), jnp.float32),
            pltpu.SemaphoreType.DMA,
        ],
    )
    def seg_kernel(h_hbm, src_hbm, dst_hbm, z_hbm, out_hbm,
                   src_v, dst_v, rows_v, acc, sem):
        c = lax.axis_index("c")
        s = lax.axis_index("s")
        wid = c * NS + s
        base = s * rows_per
        pltpu.sync_copy(src_hbm.at[wid], src_v)
        pltpu.sync_copy(dst_hbm.at[wid], dst_v)
        pltpu.sync_copy(z_hbm.at[pl.ds(base, rows_per)], acc.at[pl.ds(base, rows_per)])
        plsc.subcore_barrier()

        def body(j, carry):
            pltpu.async_copy(h_hbm.at[src_v.at[j]], rows_v, sem).wait()
            pltpu.sync_copy(rows_v, acc.at[dst_v.at[j]], add=True)
            return carry

        lax.fori_loop(0, nb, body, 0)
        plsc.subcore_barrier()
        pltpu.sync_copy(acc.at[pl.ds(base, rows_per)], out_hbm.at[c, pl.ds(base, rows_per)])

    return seg_kernel(h, src3, dst3, zeros_rows)


def _dinv_tc(deg_parts):
    r_pad = deg_parts.shape[1]

    def body(deg_ref, out_ref):
        deg = deg_ref[0:1, :] + deg_ref[1:2, :]
        out_ref[...] = lax.rsqrt(jnp.maximum(deg, 1.0))

    return pl.pallas_call(
        body,
        out_shape=jax.ShapeDtypeStruct((1, r_pad), jnp.float32),
    )(deg_parts)


def _scale_tc(feat, dinv_col):
    r_pad, d = feat.shape
    rb = r_pad // NS
    grid = (NS,)

    def body(f_ref, w_ref, o_ref):
        o_ref[...] = f_ref[...] * w_ref[...]

    return pl.pallas_call(
        body,
        grid=grid,
        in_specs=[
            pl.BlockSpec((rb, d), lambda i: (i, 0)),
            pl.BlockSpec((rb, 1), lambda i: (i, 0)),
        ],
        out_specs=pl.BlockSpec((rb, d), lambda i: (i, 0)),
        out_shape=jax.ShapeDtypeStruct((r_pad, d), jnp.float32),
    )(feat, dinv_col)


def _combine_tc(feat, agg_parts, dinv_col):
    r_pad, d = feat.shape
    rb = r_pad // NS
    grid = (NS,)

    def body(f_ref, a_ref, w_ref, f1_ref, h2_ref):
        a = a_ref[...]
        agg = a[0] + a[1]
        w = w_ref[...]
        f1 = f_ref[...] - agg * w
        f1_ref[...] = f1
        h2_ref[...] = f1 * w

    return pl.pallas_call(
        body,
        grid=grid,
        in_specs=[
            pl.BlockSpec((rb, d), lambda i: (i, 0)),
            pl.BlockSpec((NC, rb, d), lambda i: (0, i, 0)),
            pl.BlockSpec((rb, 1), lambda i: (i, 0)),
        ],
        out_specs=[
            pl.BlockSpec((rb, d), lambda i: (i, 0)),
            pl.BlockSpec((rb, d), lambda i: (i, 0)),
        ],
        out_shape=[
            jax.ShapeDtypeStruct((r_pad, d), jnp.float32),
            jax.ShapeDtypeStruct((r_pad, d), jnp.float32),
        ],
    )(feat, agg_parts, dinv_col)


def _final_tc(feat, f1, agg_parts, dinv_col):
    r_pad, d = feat.shape
    rb = r_pad // NS
    grid = (NS,)

    def body(f_ref, f1_ref, a_ref, w_ref, o_ref):
        a = a_ref[...]
        agg = a[0] + a[1]
        o_ref[...] = (T0 * f_ref[...] + (T1 + T2) * f1_ref[...]
                      - T2 * agg * w_ref[...])

    return pl.pallas_call(
        body,
        grid=grid,
        in_specs=[
            pl.BlockSpec((rb, d), lambda i: (i, 0)),
            pl.BlockSpec((rb, d), lambda i: (i, 0)),
            pl.BlockSpec((NC, rb, d), lambda i: (0, i, 0)),
            pl.BlockSpec((rb, 1), lambda i: (i, 0)),
        ],
        out_specs=pl.BlockSpec((rb, d), lambda i: (i, 0)),
        out_shape=jax.ShapeDtypeStruct((r_pad, d), jnp.float32),
    )(feat, f1, agg_parts, dinv_col)


def kernel(feat, edge_index):
    n, d = feat.shape
    e = edge_index.shape[1]
    nb = -(-e // (NW * K))
    ep = NW * nb * K
    r_pad = ((n + 16 + 2047) // 2048) * 2048
    dummy = n

    src = edge_index[0]
    dst = edge_index[1]
    pad_idx = jnp.full((ep - e,), dummy, jnp.int32)
    src3 = jnp.concatenate([src, pad_idx]).reshape(NW, nb, K)
    dst3 = jnp.concatenate([dst, pad_idx]).reshape(NW, nb, K)
    feat_p = jnp.zeros((r_pad, d), feat.dtype).at[:n].set(feat)
    zeros_rows = jnp.zeros((r_pad, d), jnp.float32)
    zeros_vec = jnp.zeros((r_pad,), jnp.float32)
    ones_vec = jnp.ones((K,), jnp.float32)

    deg_parts = jnp.reshape(_degree_sc(src3, zeros_vec, ones_vec, r_pad), (NC, r_pad))
    dinv_col = jnp.reshape(_dinv_tc(deg_parts), (r_pad, 1))
    h1 = _scale_tc(feat_p, dinv_col)
    agg1 = _segsum_sc(h1, src3, dst3, zeros_rows, r_pad)
    f1, h2 = _combine_tc(feat_p, agg1, dinv_col)
    agg2 = _segsum_sc(h2, src3, dst3, zeros_rows, r_pad)
    out = _final_tc(feat_p, f1, agg2, dinv_col)
    return out[:n]
